# baseline (speedup 1.0000x reference)
import functools

import jax
import jax.numpy as jnp
from jax.experimental import pallas as pl
from jax.experimental.pallas import tpu as pltpu

N_EXPERTS = 14
TOP_K = 2
LANES = 128
BLOCK_T = 1024


def _router_kernel(h_ref, wt_ref, bp_ref,
                   wi_out, loss_out, pair_out, probs_out,
                   acc_ref, *, n_tokens, n_steps):
    step = pl.program_id(0)

    x = h_ref[...]
    logits = jnp.dot(x, wt_ref[...], preferred_element_type=jnp.float32)
    logits = logits + bp_ref[...]

    m = jnp.max(logits, axis=-1, keepdims=True)
    e = jnp.exp(logits - m)
    s = jnp.sum(e, axis=-1, keepdims=True)
    p = e / s
    probs_out[...] = p[:, :N_EXPERTS]

    iota = jax.lax.broadcasted_iota(jnp.int32, p.shape, 1)
    m1 = jnp.max(p, axis=-1, keepdims=True)
    i1 = jnp.min(jnp.where(p == m1, iota, LANES), axis=-1, keepdims=True)
    p2 = jnp.where(iota == i1, -1.0, p)
    m2 = jnp.max(p2, axis=-1, keepdims=True)
    i2 = jnp.min(jnp.where(p2 == m2, iota, LANES), axis=-1, keepdims=True)

    denom = jnp.maximum(m1 + m2, 1e-10)
    wi_out[...] = jnp.concatenate(
        [m1 / denom, m2 / denom,
         i1.astype(jnp.float32), i2.astype(jnp.float32)], axis=1)

    hit = (i1 // 2) == (i2 // 2)
    pair_idx = i1 // 2
    onehot = jnp.where((iota == pair_idx) & hit, 1.0, 0.0)

    usage_cnt = jnp.sum(p, axis=0, keepdims=True)
    pair_cnt = jnp.sum(onehot, axis=0, keepdims=True)
    block_acc = jnp.concatenate([usage_cnt, pair_cnt], axis=0)

    @pl.when(step == 0)
    def _init():
        acc_ref[...] = block_acc

    @pl.when(step != 0)
    def _accum():
        acc_ref[...] = acc_ref[...] + block_acc

    @pl.when(step == n_steps - 1)
    def _finalize():
        usage = acc_ref[0:1, :] / n_tokens
        loss_out[...] = (N_EXPERTS * jnp.sum(usage * usage)).reshape(1, 1)
        pair_out[...] = acc_ref[1:2, :] / n_tokens


def kernel(h, W_gate, b_gate):
    B, T, D = h.shape
    n_tokens = B * T
    n_steps = n_tokens // BLOCK_T

    hf = h.reshape(n_tokens, D)
    wt = jnp.zeros((D, LANES), jnp.float32).at[:, :N_EXPERTS].set(W_gate.T)
    bp = jnp.full((1, LANES), -1e30, jnp.float32).at[0, :N_EXPERTS].set(b_gate)

    wi, loss, pair, probs = pl.pallas_call(
        functools.partial(_router_kernel, n_tokens=n_tokens, n_steps=n_steps),
        grid=(n_steps,),
        in_specs=[
            pl.BlockSpec((BLOCK_T, D), lambda i: (i, 0)),
            pl.BlockSpec((D, LANES), lambda i: (0, 0)),
            pl.BlockSpec((1, LANES), lambda i: (0, 0)),
        ],
        out_specs=[
            pl.BlockSpec((BLOCK_T, 2 * TOP_K), lambda i: (i, 0)),
            pl.BlockSpec((1, 1), lambda i: (0, 0)),
            pl.BlockSpec((1, LANES), lambda i: (0, 0)),
            pl.BlockSpec((BLOCK_T, N_EXPERTS), lambda i: (i, 0)),
        ],
        out_shape=[
            jax.ShapeDtypeStruct((n_tokens, 2 * TOP_K), jnp.float32),
            jax.ShapeDtypeStruct((1, 1), jnp.float32),
            jax.ShapeDtypeStruct((1, LANES), jnp.float32),
            jax.ShapeDtypeStruct((n_tokens, N_EXPERTS), jnp.float32),
        ],
        scratch_shapes=[pltpu.VMEM((2, LANES), jnp.float32)],
        compiler_params=pltpu.CompilerParams(
            dimension_semantics=("arbitrary",),
        ),
    )(hf, wt, bp)

    return (
        wi[:, :TOP_K].reshape(B, T, TOP_K),
        wi[:, TOP_K:].astype(jnp.int32).reshape(B, T, TOP_K),
        loss[0, 0],
        pair[0, :N_EXPERTS // 2],
        probs.reshape(B, T, N_EXPERTS),
    )


# speedup vs baseline: 1.0803x; 1.0803x over previous
import functools

import jax
import jax.numpy as jnp
from jax.experimental import pallas as pl
from jax.experimental.pallas import tpu as pltpu

N_EXPERTS = 14
TOP_K = 2
LANES = 128
BLOCK_T = 1024


def _router_kernel(h_ref, wt_ref, bp_ref,
                   w_out, i_out, loss_out, pair_out, probs_out,
                   acc_ref, *, n_tokens, n_steps):
    step = pl.program_id(0)

    x = h_ref[...]
    logits = jnp.dot(x, wt_ref[...], preferred_element_type=jnp.float32)
    logits = logits + bp_ref[...]

    m = jnp.max(logits, axis=-1, keepdims=True)
    e = jnp.exp(logits - m)
    s = jnp.sum(e, axis=-1, keepdims=True)
    p = e / s
    probs_out[...] = p[:, :N_EXPERTS]

    iota = jax.lax.broadcasted_iota(jnp.int32, p.shape, 1)
    m1 = jnp.max(p, axis=-1, keepdims=True)
    i1 = jnp.min(jnp.where(p == m1, iota, LANES), axis=-1, keepdims=True)
    p2 = jnp.where(iota == i1, -1.0, p)
    m2 = jnp.max(p2, axis=-1, keepdims=True)
    i2 = jnp.min(jnp.where(p2 == m2, iota, LANES), axis=-1, keepdims=True)

    denom = jnp.maximum(m1 + m2, 1e-10)
    w_out[...] = jnp.concatenate([m1 / denom, m2 / denom], axis=1)
    i_out[...] = jnp.concatenate([i1, i2], axis=1)

    hit = (i1 // 2) == (i2 // 2)
    pair_idx = i1 // 2
    onehot = jnp.where((iota == pair_idx) & hit, 1.0, 0.0)

    usage_cnt = jnp.sum(p, axis=0, keepdims=True)
    pair_cnt = jnp.sum(onehot, axis=0, keepdims=True)
    block_acc = jnp.concatenate([usage_cnt, pair_cnt], axis=0)

    @pl.when(step == 0)
    def _init():
        acc_ref[...] = block_acc

    @pl.when(step != 0)
    def _accum():
        acc_ref[...] = acc_ref[...] + block_acc

    @pl.when(step == n_steps - 1)
    def _finalize():
        usage = acc_ref[0:1, :] / n_tokens
        loss_out[...] = (N_EXPERTS * jnp.sum(usage * usage)).reshape(1, 1)
        pair_out[...] = acc_ref[1:2, :] / n_tokens


def kernel(h, W_gate, b_gate):
    B, T, D = h.shape
    n_tokens = B * T
    n_steps = n_tokens // BLOCK_T

    hf = h.reshape(n_tokens, D)
    wt = jnp.zeros((D, LANES), jnp.float32).at[:, :N_EXPERTS].set(W_gate.T)
    bp = jnp.full((1, LANES), -1e30, jnp.float32).at[0, :N_EXPERTS].set(b_gate)

    w, i, loss, pair, probs = pl.pallas_call(
        functools.partial(_router_kernel, n_tokens=n_tokens, n_steps=n_steps),
        grid=(n_steps,),
        in_specs=[
            pl.BlockSpec((BLOCK_T, D), lambda i: (i, 0)),
            pl.BlockSpec((D, LANES), lambda i: (0, 0)),
            pl.BlockSpec((1, LANES), lambda i: (0, 0)),
        ],
        out_specs=[
            pl.BlockSpec((BLOCK_T, TOP_K), lambda i: (i, 0)),
            pl.BlockSpec((BLOCK_T, TOP_K), lambda i: (i, 0)),
            pl.BlockSpec((1, 1), lambda i: (0, 0)),
            pl.BlockSpec((1, LANES), lambda i: (0, 0)),
            pl.BlockSpec((BLOCK_T, N_EXPERTS), lambda i: (i, 0)),
        ],
        out_shape=[
            jax.ShapeDtypeStruct((n_tokens, TOP_K), jnp.float32),
            jax.ShapeDtypeStruct((n_tokens, TOP_K), jnp.int32),
            jax.ShapeDtypeStruct((1, 1), jnp.float32),
            jax.ShapeDtypeStruct((1, LANES), jnp.float32),
            jax.ShapeDtypeStruct((n_tokens, N_EXPERTS), jnp.float32),
        ],
        scratch_shapes=[pltpu.VMEM((2, LANES), jnp.float32)],
        compiler_params=pltpu.CompilerParams(
            dimension_semantics=("arbitrary",),
        ),
    )(hf, wt, bp)

    return (
        w.reshape(B, T, TOP_K),
        i.reshape(B, T, TOP_K),
        loss[0, 0],
        pair[0, :N_EXPERTS // 2],
        probs.reshape(B, T, N_EXPERTS),
    )
